# baseline (speedup 1.0000x reference)
import functools

import jax
import jax.numpy as jnp
from jax import lax
from jax.experimental import pallas as pl
from jax.experimental.pallas import tpu as pltpu
from jax.experimental.pallas import tpu_sc as plsc

EMB = 128
B_BLOCK = 512


def _sc_gather_body(m_tab_hbm, d_tab_hbm, mi_hbm, di_hbm, om_hbm, od_hbm,
                    idx_v, rows_v, sem, *, n_cores, b_per_w):
    wid = lax.axis_index("s") * n_cores + lax.axis_index("c")
    base = wid * b_per_w
    pltpu.sync_copy(mi_hbm.at[pl.ds(base, b_per_w)], idx_v)
    pltpu.async_copy(m_tab_hbm.at[idx_v], rows_v, sem).wait()
    pltpu.sync_copy(rows_v, om_hbm.at[pl.ds(base, b_per_w)])
    pltpu.sync_copy(di_hbm.at[pl.ds(base, b_per_w)], idx_v)
    pltpu.async_copy(d_tab_hbm.at[idx_v], rows_v, sem).wait()
    pltpu.sync_copy(rows_v, od_hbm.at[pl.ds(base, b_per_w)])


def _sc_gather(model_emb_w, decoding_emb_w, model_ids, decoding_ids):
    B = model_ids.shape[0]
    info = plsc.get_sparse_core_info()
    nw = info.num_cores * info.num_subcores
    b_per_w = B // nw
    mesh = plsc.VectorSubcoreMesh(core_axis_name="c", subcore_axis_name="s")
    k = pl.kernel(
        functools.partial(_sc_gather_body, n_cores=info.num_cores,
                          b_per_w=b_per_w),
        mesh=mesh,
        out_type=[
            jax.ShapeDtypeStruct((B, EMB), jnp.float32),
            jax.ShapeDtypeStruct((B, EMB), jnp.float32),
        ],
        scratch_types=[
            pltpu.VMEM((b_per_w,), jnp.int32),
            pltpu.VMEM((b_per_w, EMB), jnp.float32),
            pltpu.SemaphoreType.DMA,
        ],
    )
    return k(model_emb_w, decoding_emb_w, model_ids, decoding_ids)


def _mlp_body(x_ref, m_ref, d_ref, qw_ref, qb_ref, w1_ref, b1_ref, w2_ref,
              b2_ref, out_ref, weff_ref, beff_ref):
    @pl.when(pl.program_id(0) == 0)
    def _fold():
        weff_ref[...] = w1_ref[:, : 3 * EMB]
        weff_ref[:, 2 * EMB: 3 * EMB] += lax.dot_general(
            w1_ref[:, 5 * EMB: 6 * EMB], qw_ref[...],
            (((1,), (0,)), ((), ())), preferred_element_type=jnp.float32)
        beff_ref[...] = b1_ref[...] + lax.dot_general(
            qb_ref[...], w1_ref[:, 5 * EMB: 6 * EMB],
            (((1,), (1,)), ((), ())), preferred_element_type=jnp.float32)

    acc = lax.dot_general(x_ref[...], weff_ref[...],
                          (((1,), (1,)), ((), ())),
                          preferred_element_type=jnp.float32)
    acc += lax.dot_general(m_ref[...], w1_ref[:, 3 * EMB: 4 * EMB],
                           (((1,), (1,)), ((), ())),
                           preferred_element_type=jnp.float32)
    acc += lax.dot_general(d_ref[...], w1_ref[:, 4 * EMB: 5 * EMB],
                           (((1,), (1,)), ((), ())),
                           preferred_element_type=jnp.float32)
    acc += beff_ref[...]
    h1 = jnp.maximum(acc, 0.0)
    out_ref[...] = lax.dot_general(h1, w2_ref[...],
                                   (((1,), (1,)), ((), ())),
                                   preferred_element_type=jnp.float32) \
        + b2_ref[...]


def _mlp_tc(x, m_emb, d_emb, q_w, q_b, w1, b1, w2, b2):
    B, in_dim = x.shape
    hidden = w1.shape[0]
    ncls = w2.shape[0]
    grid = (B // B_BLOCK,)
    return pl.pallas_call(
        _mlp_body,
        grid=grid,
        in_specs=[
            pl.BlockSpec((B_BLOCK, in_dim), lambda i: (i, 0)),
            pl.BlockSpec((B_BLOCK, EMB), lambda i: (i, 0)),
            pl.BlockSpec((B_BLOCK, EMB), lambda i: (i, 0)),
            pl.BlockSpec((EMB, EMB), lambda i: (0, 0)),
            pl.BlockSpec((1, EMB), lambda i: (0, 0)),
            pl.BlockSpec((hidden, in_dim + 3 * EMB), lambda i: (0, 0)),
            pl.BlockSpec((1, hidden), lambda i: (0, 0)),
            pl.BlockSpec((ncls, hidden), lambda i: (0, 0)),
            pl.BlockSpec((1, ncls), lambda i: (0, 0)),
        ],
        out_specs=pl.BlockSpec((B_BLOCK, ncls), lambda i: (i, 0)),
        out_shape=jax.ShapeDtypeStruct((B, ncls), jnp.float32),
        scratch_shapes=[
            pltpu.VMEM((hidden, in_dim), jnp.float32),
            pltpu.VMEM((1, hidden), jnp.float32),
        ],
        compiler_params=pltpu.CompilerParams(
            dimension_semantics=("arbitrary",)),
    )(x, m_emb, d_emb, q_w, q_b.reshape(1, EMB), w1, b1.reshape(1, hidden),
      w2, b2.reshape(1, ncls))


def kernel(x, model_ids, decoding_ids, model_emb_w, decoding_emb_w, q_w, q_b,
           w1, b1, w2, b2):
    m_emb, d_emb = _sc_gather(model_emb_w, decoding_emb_w,
                              model_ids.astype(jnp.int32),
                              decoding_ids.astype(jnp.int32))
    return _mlp_tc(x, m_emb, d_emb, q_w, q_b, w1, b1, w2, b2)


# speedup vs baseline: 1.0058x; 1.0058x over previous
import functools

import jax
import jax.numpy as jnp
from jax import lax
from jax.experimental import pallas as pl
from jax.experimental.pallas import tpu as pltpu
from jax.experimental.pallas import tpu_sc as plsc

EMB = 128
B_BLOCK = 512
CHUNK = 256


def _sc_gather_body(m_tab_hbm, d_tab_hbm, mi_hbm, di_hbm, om_hbm, od_hbm,
                    i0, i1, i2, i3, b0, b1, b2, s0, s1, s2,
                    *, n_cores, b_per_w):
    wid = lax.axis_index("s") * n_cores + lax.axis_index("c")
    base = wid * b_per_w
    pltpu.sync_copy(mi_hbm.at[pl.ds(base, CHUNK)], i0)
    pltpu.sync_copy(mi_hbm.at[pl.ds(base + CHUNK, CHUNK)], i1)
    pltpu.sync_copy(di_hbm.at[pl.ds(base, CHUNK)], i2)
    pltpu.sync_copy(di_hbm.at[pl.ds(base + CHUNK, CHUNK)], i3)
    c0 = pltpu.async_copy(m_tab_hbm.at[i0], b0, s0)
    c1 = pltpu.async_copy(m_tab_hbm.at[i1], b1, s1)
    c2 = pltpu.async_copy(d_tab_hbm.at[i2], b2, s2)
    c0.wait()
    pltpu.sync_copy(b0, om_hbm.at[pl.ds(base, CHUNK)])
    c3 = pltpu.async_copy(d_tab_hbm.at[i3], b0, s0)
    c1.wait()
    pltpu.sync_copy(b1, om_hbm.at[pl.ds(base + CHUNK, CHUNK)])
    c2.wait()
    pltpu.sync_copy(b2, od_hbm.at[pl.ds(base, CHUNK)])
    c3.wait()
    pltpu.sync_copy(b0, od_hbm.at[pl.ds(base + CHUNK, CHUNK)])


def _sc_gather(model_emb_w, decoding_emb_w, model_ids, decoding_ids):
    B = model_ids.shape[0]
    info = plsc.get_sparse_core_info()
    nw = info.num_cores * info.num_subcores
    b_per_w = B // nw
    mesh = plsc.VectorSubcoreMesh(core_axis_name="c", subcore_axis_name="s")
    k = pl.kernel(
        functools.partial(_sc_gather_body, n_cores=info.num_cores,
                          b_per_w=b_per_w),
        mesh=mesh,
        out_type=[
            jax.ShapeDtypeStruct((B, EMB), jnp.float32),
            jax.ShapeDtypeStruct((B, EMB), jnp.float32),
        ],
        scratch_types=[
            pltpu.VMEM((CHUNK,), jnp.int32),
            pltpu.VMEM((CHUNK,), jnp.int32),
            pltpu.VMEM((CHUNK,), jnp.int32),
            pltpu.VMEM((CHUNK,), jnp.int32),
            pltpu.VMEM((CHUNK, EMB), jnp.float32),
            pltpu.VMEM((CHUNK, EMB), jnp.float32),
            pltpu.VMEM((CHUNK, EMB), jnp.float32),
            pltpu.SemaphoreType.DMA,
            pltpu.SemaphoreType.DMA,
            pltpu.SemaphoreType.DMA,
        ],
    )
    return k(model_emb_w, decoding_emb_w, model_ids, decoding_ids)


def _mlp_body(x_ref, m_ref, d_ref, qw_ref, qb_ref, w1_ref, b1_ref, w2_ref,
              b2_ref, out_ref, weff_ref, beff_ref):
    @pl.when(pl.program_id(0) == 0)
    def _fold():
        weff_ref[...] = w1_ref[:, : 3 * EMB]
        weff_ref[:, 2 * EMB: 3 * EMB] += lax.dot_general(
            w1_ref[:, 5 * EMB: 6 * EMB], qw_ref[...],
            (((1,), (0,)), ((), ())),
            preferred_element_type=jnp.float32).astype(jnp.bfloat16)
        beff_ref[...] = b1_ref[...] + lax.dot_general(
            qb_ref[...], w1_ref[:, 5 * EMB: 6 * EMB].astype(jnp.float32),
            (((1,), (1,)), ((), ())), preferred_element_type=jnp.float32)

    xb = x_ref[...].astype(jnp.bfloat16)
    acc = lax.dot_general(xb, weff_ref[...],
                          (((1,), (1,)), ((), ())),
                          preferred_element_type=jnp.float32)
    acc += lax.dot_general(m_ref[...].astype(jnp.bfloat16),
                           w1_ref[:, 3 * EMB: 4 * EMB],
                           (((1,), (1,)), ((), ())),
                           preferred_element_type=jnp.float32)
    acc += lax.dot_general(d_ref[...].astype(jnp.bfloat16),
                           w1_ref[:, 4 * EMB: 5 * EMB],
                           (((1,), (1,)), ((), ())),
                           preferred_element_type=jnp.float32)
    acc += beff_ref[...]
    h1 = jnp.maximum(acc, 0.0).astype(jnp.bfloat16)
    out_ref[...] = lax.dot_general(h1, w2_ref[...],
                                   (((1,), (1,)), ((), ())),
                                   preferred_element_type=jnp.float32) \
        + b2_ref[...]


def _mlp_tc(x, m_emb, d_emb, q_w, q_b, w1, b1, w2, b2):
    B, in_dim = x.shape
    hidden = w1.shape[0]
    ncls = w2.shape[0]
    grid = (B // B_BLOCK,)
    return pl.pallas_call(
        _mlp_body,
        grid=grid,
        in_specs=[
            pl.BlockSpec((B_BLOCK, in_dim), lambda i: (i, 0)),
            pl.BlockSpec((B_BLOCK, EMB), lambda i: (i, 0)),
            pl.BlockSpec((B_BLOCK, EMB), lambda i: (i, 0)),
            pl.BlockSpec((EMB, EMB), lambda i: (0, 0)),
            pl.BlockSpec((1, EMB), lambda i: (0, 0)),
            pl.BlockSpec((hidden, in_dim + 3 * EMB), lambda i: (0, 0)),
            pl.BlockSpec((1, hidden), lambda i: (0, 0)),
            pl.BlockSpec((ncls, hidden), lambda i: (0, 0)),
            pl.BlockSpec((1, ncls), lambda i: (0, 0)),
        ],
        out_specs=pl.BlockSpec((B_BLOCK, ncls), lambda i: (i, 0)),
        out_shape=jax.ShapeDtypeStruct((B, ncls), jnp.float32),
        scratch_shapes=[
            pltpu.VMEM((hidden, in_dim), jnp.bfloat16),
            pltpu.VMEM((1, hidden), jnp.float32),
        ],
        compiler_params=pltpu.CompilerParams(
            dimension_semantics=("arbitrary",)),
    )(x, m_emb, d_emb, q_w, q_b.reshape(1, EMB), w1, b1.reshape(1, hidden),
      w2, b2.reshape(1, ncls))


def kernel(x, model_ids, decoding_ids, model_emb_w, decoding_emb_w, q_w, q_b,
           w1, b1, w2, b2):
    bf = jnp.bfloat16
    m_emb, d_emb = _sc_gather(model_emb_w, decoding_emb_w,
                              model_ids.astype(jnp.int32),
                              decoding_ids.astype(jnp.int32))
    return _mlp_tc(x, m_emb, d_emb, q_w.astype(bf), q_b, w1.astype(bf), b1,
                   w2.astype(bf), b2)
